# Initial kernel scaffold; baseline (speedup 1.0000x reference)
import jax
import jax.numpy as jnp
from jax.experimental import pallas as pl


def kernel(a_velocity_length, a_velocity_theta, a_length, a_width, a_type, position, heading, visible_mask, l_embs, params):
    raise NotImplementedError("write your pallas kernel here")


# speedup vs baseline: 350.8722x; 350.8722x over previous
import functools

import jax
import jax.numpy as jnp
from jax import lax
from jax.experimental import pallas as pl
from jax.experimental.pallas import tpu as pltpu

N_AGENTS = 512
T_HIST = 20
T_FUT = 30
K_MODES = 6
DURATION = 10
D = 128
H = 8
HD = D // H

AGENTS_PER_BLOCK = 32
R = AGENTS_PER_BLOCK * T_HIST
N_BLOCKS = N_AGENTS // AGENTS_PER_BLOCK
OUT_W = K_MODES * T_FUT * 2


def _shift_down(x, d):
    if d == 0:
        return x
    pad = jnp.zeros((d, x.shape[1]), x.dtype)
    return jnp.concatenate([pad, x[: x.shape[0] - d, :]], axis=0)


def _wrap_angle(a):
    return (a + jnp.pi) % (2.0 * jnp.pi) - jnp.pi


def _body(vl, vt, al, aw, aty, px, py, hd, mvis,
          mt, aw1, ab1, aw2, ab2,
          ew1, eb1, ew2, eb2,
          wq, wk, wv, we, wo,
          tw1, tb1, tw2, tb2,
          out_ref):
    f32 = jnp.float32

    def mm(a, b):
        return jnp.dot(a, b, preferred_element_type=f32)

    g = (vl[...] * aw1[0:1, :] + vt[...] * aw1[1:2, :] + al[...] * aw1[2:3, :]
         + aw[...] * aw1[3:4, :] + aty[...] * aw1[4:5, :] + ab1[...])
    g = jnp.maximum(g, 0.0)
    te = mm(g, aw2[...]) + ab2[...]

    k_all = mm(te, wk[...])
    v_all = mm(te, wv[...])

    q8 = mm(mt[...], wq[...])
    qt = q8.T
    ci = lax.broadcasted_iota(jnp.int32, (D, H), 0)
    hi = lax.broadcasted_iota(jnp.int32, (D, H), 1)
    head_mask = (ci // HD == hi).astype(f32)
    m_mat = jnp.concatenate(
        [qt[:, k:k + 1] * head_mask for k in range(K_MODES)], axis=1)
    hi2 = lax.broadcasted_iota(jnp.int32, (H, D), 0)
    ci2 = lax.broadcasted_iota(jnp.int32, (H, D), 1)
    spread = (ci2 // HD == hi2).astype(f32)

    tpr = lax.broadcasted_iota(jnp.int32, (R, 1), 0) % T_HIST
    c_h = jnp.cos(hd[...])
    s_h = jnp.sin(hd[...])

    logits = []
    ve = []
    for d in range(DURATION + 1):
        px_s = _shift_down(px[...], d)
        py_s = _shift_down(py[...], d)
        hd_s = _shift_down(hd[...], d)
        vx = px_s - px[...]
        vy = py_s - py[...]
        lx = c_h * vx + s_h * vy
        ly = -s_h * vx + c_h * vy
        length = jnp.sqrt(lx * lx + ly * ly)
        theta = jnp.arctan2(ly, lx)
        rel = _wrap_angle(hd_s - hd[...])
        ge = (length * ew1[0:1, :] + theta * ew1[1:2, :] + rel * ew1[2:3, :]
              + (-float(d)) * ew1[3:4, :] + eb1[...])
        ge = jnp.maximum(ge, 0.0)
        ea = mm(ge, ew2[...]) + eb2[...]
        evalid = _shift_down(mvis[...], d) * mvis[...]
        ea = ea * evalid
        eh = mm(ea, we[...])
        l_d = mm(_shift_down(k_all, d) + eh, m_mat) * (1.0 / 4.0)
        l_d = jnp.where(tpr >= d, l_d, -1e30)
        logits.append(l_d)
        ve.append(_shift_down(v_all, d) + eh)

    mx = functools.reduce(jnp.maximum, logits)
    probs = [jnp.exp(l - mx) for l in logits]
    den = functools.reduce(lambda a, b: a + b, probs)

    outs = []
    for k in range(K_MODES):
        acc = jnp.zeros((R, D), f32)
        for d in range(DURATION + 1):
            a_dk = probs[d][:, k * H:(k + 1) * H]
            acc = acc + mm(a_dk, spread) * ve[d]
        denk = mm(den[:, k * H:(k + 1) * H], spread)
        agg = acc / (denk + 1e-9)
        mo = mt[k:k + 1, :] + mm(agg, wo[...])
        th = jnp.maximum(mm(mo, tw1[...]) + tb1[...], 0.0)
        outs.append(mm(th, tw2[...]) + tb2[...])

    out_ref[...] = jnp.concatenate(outs, axis=1)


def kernel(a_velocity_length, a_velocity_theta, a_length, a_width, a_type,
           position, heading, visible_mask, l_embs, params):
    f32 = jnp.float32
    NT = N_AGENTS * T_HIST

    col = lambda x: x.reshape(NT, 1).astype(f32)
    vl = col(a_velocity_length)
    vt = col(a_velocity_theta)
    al = col(jnp.repeat(a_length[:, None], T_HIST, axis=1))
    aw = col(jnp.repeat(a_width[:, None], T_HIST, axis=1))
    aty = col(jnp.repeat(a_type[:, None], T_HIST, axis=1))
    px = col(position[:, :, 0])
    py = col(position[:, :, 1])
    hd = col(heading)
    mvis = col(visible_mask)

    p = params
    mt = jnp.zeros((8, D), f32).at[:K_MODES].set(p['mode_tokens'])
    row = lambda b: b.reshape(1, -1).astype(f32)
    weights = [
        mt,
        p['a_emb']['w1'], row(p['a_emb']['b1']),
        p['a_emb']['w2'], row(p['a_emb']['b2']),
        p['t2m_emb']['w1'], row(p['t2m_emb']['b1']),
        p['t2m_emb']['w2'], row(p['t2m_emb']['b2']),
        p['t2m_attn']['wq'], p['t2m_attn']['wk'], p['t2m_attn']['wv'],
        p['t2m_attn']['we'], p['t2m_attn']['wo'],
        p['traj_propose']['w1'], row(p['traj_propose']['b1']),
        p['traj_propose']['w2'], row(p['traj_propose']['b2']),
    ]

    row_spec = pl.BlockSpec((R, 1), lambda i: (i, 0))
    full = lambda a: pl.BlockSpec(a.shape, lambda i: (0,) * a.ndim)

    out = pl.pallas_call(
        _body,
        grid=(N_BLOCKS,),
        in_specs=[row_spec] * 9 + [full(w) for w in weights],
        out_specs=pl.BlockSpec((R, OUT_W), lambda i: (i, 0)),
        out_shape=jax.ShapeDtypeStruct((NT, OUT_W), f32),
        compiler_params=pltpu.CompilerParams(
            dimension_semantics=("arbitrary",)),
    )(vl, vt, al, aw, aty, px, py, hd, mvis, *weights)

    return out.reshape(N_AGENTS, T_HIST, K_MODES, T_FUT, 2)
